# initial kernel scaffold (unmeasured)
import jax
import jax.numpy as jnp
from jax import lax
from jax.experimental import pallas as pl
from jax.experimental.pallas import tpu as pltpu


def kernel(partial, resid, gamma):
    m, d = partial.shape[1], partial.shape[2]
    p = partial.reshape(m, d)
    g = gamma.reshape(1, d)

    def body(p_ref, resid_ref, g_ref, out_ref, comm_ref, send_sem, recv_sem):
        my_x = lax.axis_index("x")
        my_y = lax.axis_index("y")
        nbr = (my_x, 1 - my_y)

        barrier_sem = pltpu.get_barrier_semaphore()
        pl.semaphore_signal(
            barrier_sem, inc=1, device_id=nbr,
            device_id_type=pl.DeviceIdType.MESH,
        )
        pl.semaphore_wait(barrier_sem, 1)

        rdma = pltpu.make_async_remote_copy(
            src_ref=p_ref,
            dst_ref=comm_ref,
            send_sem=send_sem,
            recv_sem=recv_sem,
            device_id=nbr,
            device_id_type=pl.DeviceIdType.MESH,
        )
        rdma.start()
        rdma.wait()

        y = p_ref[...] + comm_ref[...] + resid_ref[...]
        ms = jnp.mean(y * y, axis=-1, keepdims=True)
        out_ref[...] = y * lax.rsqrt(ms + 1e-6) * g_ref[...]

    return pl.pallas_call(
        body,
        out_shape=jax.ShapeDtypeStruct((m, d), jnp.float32),
        in_specs=[pl.BlockSpec(memory_space=pltpu.VMEM)] * 3,
        out_specs=pl.BlockSpec(memory_space=pltpu.VMEM),
        scratch_shapes=[
            pltpu.VMEM((m, d), jnp.float32),
            pltpu.SemaphoreType.DMA,
            pltpu.SemaphoreType.DMA,
        ],
        compiler_params=pltpu.CompilerParams(collective_id=0),
    )(p, resid, g)


# baseline (device time: 191484 ns/iter reference)
import jax
import jax.numpy as jnp
from jax import lax
from jax.experimental import pallas as pl
from jax.experimental.pallas import tpu as pltpu

K = 8


def kernel(partial, resid, gamma):
    m, d = partial.shape[1], partial.shape[2]
    cm = m // K
    p = partial.reshape(m, d)
    g = gamma.reshape(1, d)

    def body(p_ref, resid_ref, g_ref, out_ref,
             y_recv, p_buf, r_buf, o_buf,
             y_send_sems, y_recv_sems, p_sems, r_sems, o_sems):
        my_x = lax.axis_index("x")
        my_y = lax.axis_index("y")
        nbr = (my_x, 1 - my_y)

        barrier_sem = pltpu.get_barrier_semaphore()
        pl.semaphore_signal(
            barrier_sem, inc=1, device_id=nbr,
            device_id_type=pl.DeviceIdType.MESH,
        )
        pl.semaphore_wait(barrier_sem, 1)

        y_rdmas = []
        for k in range(K):
            r = pltpu.make_async_remote_copy(
                src_ref=p_ref.at[pl.ds(k * cm, cm), :],
                dst_ref=y_recv.at[k],
                send_sem=y_send_sems.at[k],
                recv_sem=y_recv_sems.at[k],
                device_id=nbr,
                device_id_type=pl.DeviceIdType.MESH,
            )
            r.start()
            y_rdmas.append(r)

        def load(k, slot):
            cp_p = pltpu.make_async_copy(
                p_ref.at[pl.ds(k * cm, cm), :], p_buf.at[slot], p_sems.at[slot])
            cp_r = pltpu.make_async_copy(
                resid_ref.at[pl.ds(k * cm, cm), :], r_buf.at[slot], r_sems.at[slot])
            cp_p.start()
            cp_r.start()
            return cp_p, cp_r

        loads = {0: load(0, 0)}
        stores = {}
        for k in range(K):
            slot = k % 2
            if k + 1 < K:
                loads[k + 1] = load(k + 1, (k + 1) % 2)
            cp_p, cp_r = loads.pop(k)
            cp_p.wait()
            cp_r.wait()
            y_rdmas[k].wait_recv()
            yk = p_buf[slot] + y_recv[k] + r_buf[slot]
            ms = jnp.mean(yk * yk, axis=-1, keepdims=True)
            if k >= 2:
                stores.pop(k - 2).wait()
            o_buf[slot] = yk * lax.rsqrt(ms + 1e-6) * g_ref[...]
            st = pltpu.make_async_copy(
                o_buf.at[slot], out_ref.at[pl.ds(k * cm, cm), :], o_sems.at[slot])
            st.start()
            stores[k] = st

        for st in stores.values():
            st.wait()
        for r in y_rdmas:
            r.wait_send()

    return pl.pallas_call(
        body,
        out_shape=jax.ShapeDtypeStruct((m, d), jnp.float32),
        in_specs=[
            pl.BlockSpec(memory_space=pl.ANY),
            pl.BlockSpec(memory_space=pl.ANY),
            pl.BlockSpec(memory_space=pltpu.VMEM),
        ],
        out_specs=pl.BlockSpec(memory_space=pl.ANY),
        scratch_shapes=[
            pltpu.VMEM((K, cm, d), jnp.float32),
            pltpu.VMEM((2, cm, d), jnp.float32),
            pltpu.VMEM((2, cm, d), jnp.float32),
            pltpu.VMEM((2, cm, d), jnp.float32),
            pltpu.SemaphoreType.DMA((K,)),
            pltpu.SemaphoreType.DMA((K,)),
            pltpu.SemaphoreType.DMA((2,)),
            pltpu.SemaphoreType.DMA((2,)),
            pltpu.SemaphoreType.DMA((2,)),
        ],
        compiler_params=pltpu.CompilerParams(collective_id=0),
    )(p, resid, g)


# device time: 109630 ns/iter; 1.7466x vs baseline; 1.7466x over previous
import jax
import jax.numpy as jnp
from jax import lax
from jax.experimental import pallas as pl
from jax.experimental.pallas import tpu as pltpu

K = 16


def kernel(partial, resid, gamma):
    m, d = partial.shape[1], partial.shape[2]
    half = m // 2
    cm = half // K
    p = partial.reshape(m, d)
    g = gamma.reshape(1, d)

    def body(p_ref, resid_ref, g_ref, out_ref,
             y_recv, x_recv, p_buf, r_buf, o_buf,
             y_send_sems, y_recv_sems, x_send_sems, x_recv_sems,
             p_sems, r_sems, o_sems, xo_sems):
        my_x = lax.axis_index("x")
        my_y = lax.axis_index("y")
        nbr_y = (my_x, 1 - my_y)
        nbr_x = (1 - my_x, my_y)
        base = my_x * half
        obase = (1 - my_x) * half

        barrier_sem = pltpu.get_barrier_semaphore()
        for nbr in (nbr_y, nbr_x):
            pl.semaphore_signal(
                barrier_sem, inc=1, device_id=nbr,
                device_id_type=pl.DeviceIdType.MESH,
            )
        pl.semaphore_wait(barrier_sem, 2)

        y_rdmas = []
        for k in range(K):
            r = pltpu.make_async_remote_copy(
                src_ref=p_ref.at[pl.ds(base + k * cm, cm), :],
                dst_ref=y_recv.at[k],
                send_sem=y_send_sems.at[k],
                recv_sem=y_recv_sems.at[k],
                device_id=nbr_y,
                device_id_type=pl.DeviceIdType.MESH,
            )
            r.start()
            y_rdmas.append(r)

        def load(k, slot):
            cp_p = pltpu.make_async_copy(
                p_ref.at[pl.ds(base + k * cm, cm), :], p_buf.at[slot],
                p_sems.at[slot])
            cp_r = pltpu.make_async_copy(
                resid_ref.at[pl.ds(base + k * cm, cm), :], r_buf.at[slot],
                r_sems.at[slot])
            cp_p.start()
            cp_r.start()
            return cp_p, cp_r

        loads = {0: load(0, 0)}
        stores = {}
        x_rdmas = {}
        for k in range(K):
            slot = k % 2
            if k + 1 < K:
                loads[k + 1] = load(k + 1, (k + 1) % 2)
            cp_p, cp_r = loads.pop(k)
            cp_p.wait()
            cp_r.wait()
            y_rdmas[k].wait_recv()
            yk = p_buf[slot] + y_recv[k] + r_buf[slot]
            ms = jnp.mean(yk * yk, axis=-1, keepdims=True)
            if k >= 2:
                stores.pop(k - 2).wait()
                x_rdmas[k - 2].wait_send()
            o_buf[slot] = yk * lax.rsqrt(ms + 1e-6) * g_ref[...]
            xr = pltpu.make_async_remote_copy(
                src_ref=o_buf.at[slot],
                dst_ref=x_recv.at[k],
                send_sem=x_send_sems.at[k],
                recv_sem=x_recv_sems.at[k],
                device_id=nbr_x,
                device_id_type=pl.DeviceIdType.MESH,
            )
            xr.start()
            x_rdmas[k] = xr
            st = pltpu.make_async_copy(
                o_buf.at[slot], out_ref.at[pl.ds(base + k * cm, cm), :],
                o_sems.at[slot])
            st.start()
            stores[k] = st

        xstores = []
        for k in range(K):
            x_rdmas[k].wait_recv()
            st = pltpu.make_async_copy(
                x_recv.at[k], out_ref.at[pl.ds(obase + k * cm, cm), :],
                xo_sems.at[k])
            st.start()
            xstores.append(st)
        for st in stores.values():
            st.wait()
        for k in (K - 2, K - 1):
            x_rdmas[k].wait_send()
        for st in xstores:
            st.wait()
        for r in y_rdmas:
            r.wait_send()

    return pl.pallas_call(
        body,
        out_shape=jax.ShapeDtypeStruct((m, d), jnp.float32),
        in_specs=[
            pl.BlockSpec(memory_space=pl.ANY),
            pl.BlockSpec(memory_space=pl.ANY),
            pl.BlockSpec(memory_space=pltpu.VMEM),
        ],
        out_specs=pl.BlockSpec(memory_space=pl.ANY),
        scratch_shapes=[
            pltpu.VMEM((K, cm, d), jnp.float32),
            pltpu.VMEM((K, cm, d), jnp.float32),
            pltpu.VMEM((2, cm, d), jnp.float32),
            pltpu.VMEM((2, cm, d), jnp.float32),
            pltpu.VMEM((2, cm, d), jnp.float32),
            pltpu.SemaphoreType.DMA((K,)),
            pltpu.SemaphoreType.DMA((K,)),
            pltpu.SemaphoreType.DMA((K,)),
            pltpu.SemaphoreType.DMA((K,)),
            pltpu.SemaphoreType.DMA((2,)),
            pltpu.SemaphoreType.DMA((2,)),
            pltpu.SemaphoreType.DMA((2,)),
            pltpu.SemaphoreType.DMA((K,)),
        ],
        compiler_params=pltpu.CompilerParams(collective_id=0),
    )(p, resid, g)
